# baseline (speedup 1.0000x reference)
import functools

import jax
import jax.numpy as jnp
from jax import lax
from jax.experimental import pallas as pl
from jax.experimental.pallas import tpu as pltpu
from jax.experimental.pallas import tpu_sc as plsc

_BATCH = 16384
_FEATURES = 640
_NNZ = 655360

_NC = 2
_NS = 16
_LANES = 16
_NW = _NC * _NS

_CHUNK = _NNZ // _NW
_ACC_SLICE = _BATCH // _NS


def _sc_partial_kernel(row_h, col_h, val_h, w_h, out_h,
                       w_v, col_v, val_v, row_v, acc_v, red_v, out_v,
                       stage_sh, sem_in, sem_z):
  cid = lax.axis_index("c")
  sid = lax.axis_index("s")
  base = (cid * _NS + sid) * _CHUNK

  cw = pltpu.async_copy(w_h.at[0], w_v, sem_in)
  cc = pltpu.async_copy(col_h.at[pl.ds(base, _CHUNK)], col_v, sem_in)
  cv = pltpu.async_copy(val_h.at[pl.ds(base, _CHUNK)], val_v, sem_in)
  cr = pltpu.async_copy(row_h.at[pl.ds(base, _CHUNK)], row_v, sem_in)

  @pl.loop(0, _BATCH // _LANES, unroll=8)
  def _zero(i):
    acc_v[pl.ds(i * _LANES, _LANES)] = jnp.zeros((_LANES,), jnp.float32)

  cw.wait()
  cc.wait()
  cv.wait()
  cr.wait()

  @pl.loop(0, _CHUNK // _LANES, unroll=2)
  def _compute(i):
    sl = pl.ds(i * _LANES, _LANES)
    cols = col_v[sl]
    wv = plsc.load_gather(w_v, [cols])
    prod = wv * val_v[sl]
    rows = row_v[sl]
    plsc.addupdate_scatter(acc_v, [rows], prod)

  pltpu.sync_copy(acc_v, stage_sh.at[sid])
  plsc.subcore_barrier()

  rdescs = [
      pltpu.async_copy(stage_sh.at[j, pl.ds(sid * _ACC_SLICE, _ACC_SLICE)],
                       red_v.at[j], sem_z)
      for j in range(_NS)
  ]
  for d in rdescs:
    d.wait()

  @pl.loop(0, _ACC_SLICE // _LANES, unroll=2)
  def _reduce(i):
    sl = pl.ds(i * _LANES, _LANES)
    s = red_v[0, sl]
    for j in range(1, _NS):
      s = s + red_v[j, sl]
    out_v[sl] = s

  pltpu.sync_copy(out_v,
                  out_h.at[cid, pl.ds(sid * _ACC_SLICE, _ACC_SLICE)])


@functools.partial(
    pl.kernel,
    out_type=jax.ShapeDtypeStruct((_NC, _BATCH), jnp.float32),
    mesh=plsc.VectorSubcoreMesh(core_axis_name="c", subcore_axis_name="s",
                                num_cores=_NC, num_subcores=_NS),
    scratch_types=[
        pltpu.VMEM((_FEATURES,), jnp.float32),
        pltpu.VMEM((_CHUNK,), jnp.int32),
        pltpu.VMEM((_CHUNK,), jnp.float32),
        pltpu.VMEM((_CHUNK,), jnp.int32),
        pltpu.VMEM((_BATCH,), jnp.float32),
        pltpu.VMEM((_NS, _ACC_SLICE), jnp.float32),
        pltpu.VMEM((_ACC_SLICE,), jnp.float32),
        pltpu.VMEM_SHARED((_NS, _BATCH), jnp.float32),
        pltpu.SemaphoreType.DMA,
        pltpu.SemaphoreType.DMA,
    ],
    compiler_params=pltpu.CompilerParams(needs_layout_passes=False),
)
def _sc_partials(row_h, col_h, val_h, w_h, out_h, *scratch):
  _sc_partial_kernel(row_h, col_h, val_h, w_h, out_h, *scratch)


def _combine_kernel(p_ref, o_ref):
  s = p_ref[0:1, :] + p_ref[1:2, :]
  o_ref[...] = jax.nn.sigmoid(s)


def kernel(row_idx, col_idx, values, W):
  partials = _sc_partials(row_idx.astype(jnp.int32),
                          col_idx.astype(jnp.int32), values, W)

  logits = pl.pallas_call(
      _combine_kernel,
      out_shape=jax.ShapeDtypeStruct((1, _BATCH), jnp.float32),
  )(partials)
  return logits.reshape(_BATCH, 1)


# speedup vs baseline: 1.2023x; 1.2023x over previous
import functools

import jax
import jax.numpy as jnp
from jax import lax
from jax.experimental import pallas as pl
from jax.experimental.pallas import tpu as pltpu
from jax.experimental.pallas import tpu_sc as plsc

_BATCH = 16384
_FEATURES = 640
_NNZ = 655360

_NC = 2
_NS = 16
_LANES = 16
_NW = _NC * _NS

_MINOR = 128
_ROWS_TOTAL = _NNZ // _MINOR
_CHUNK_R = _ROWS_TOTAL // _NW
_ACC_SLICE = _BATCH // _NS
_NBANK = 4


def _sc_partial_kernel(row_h, col_h, val_h, w_h, out_h,
                       w_v, col_v, val_v, row_v, radj_v, contrib, zero_v,
                       red_v, out_v, acc_sh, sem_in, sem_sc, sem_z):
  cid = lax.axis_index("c")
  sid = lax.axis_index("s")
  base = (cid * _NS + sid) * _CHUNK_R
  bank_off = (sid % _NBANK) * _BATCH

  cw = pltpu.async_copy(w_h.at[0], w_v, sem_in)
  cc = pltpu.async_copy(col_h.at[pl.ds(base, _CHUNK_R)], col_v, sem_in)
  cv = pltpu.async_copy(val_h.at[pl.ds(base, _CHUNK_R)], val_v, sem_in)
  cr = pltpu.async_copy(row_h.at[pl.ds(base, _CHUNK_R)], row_v, sem_in)

  @pl.loop(0, _ACC_SLICE // _LANES, unroll=4)
  def _zero(i):
    zero_v[pl.ds(i * _LANES, _LANES)] = jnp.zeros((_LANES,), jnp.float32)

  zd = [
      pltpu.async_copy(
          zero_v, acc_sh.at[pl.ds(b * _BATCH + sid * _ACC_SLICE,
                                  _ACC_SLICE)], sem_z)
      for b in range(_NBANK)
  ]
  for d in zd:
    d.wait()
  cw.wait()
  cc.wait()
  cv.wait()
  cr.wait()

  plsc.subcore_barrier()

  @pl.loop(0, _CHUNK_R)
  def _compute(r):
    for k in range(_MINOR // _LANES):
      sl = pl.ds(k * _LANES, _LANES)
      cols = col_v[r, sl]
      wv = plsc.load_gather(w_v, [cols])
      contrib[r, sl] = wv * val_v[r, sl]
      radj_v[r, sl] = row_v[r, sl] + bank_off
    pltpu.async_copy(contrib.at[r], acc_sh.at[radj_v.at[r]], sem_sc,
                     add=True)

  @pl.loop(0, _CHUNK_R)
  def _drain(r):
    pltpu.make_async_copy(contrib.at[0], acc_sh.at[radj_v.at[0]],
                          sem_sc).wait()

  plsc.subcore_barrier()

  rd = [
      pltpu.async_copy(
          acc_sh.at[pl.ds(b * _BATCH + sid * _ACC_SLICE, _ACC_SLICE)],
          red_v.at[b], sem_z)
      for b in range(_NBANK)
  ]
  for d in rd:
    d.wait()

  @pl.loop(0, _ACC_SLICE // _LANES, unroll=2)
  def _reduce(i):
    sl = pl.ds(i * _LANES, _LANES)
    s = red_v[0, sl]
    for b in range(1, _NBANK):
      s = s + red_v[b, sl]
    out_v[sl] = s

  pltpu.sync_copy(out_v,
                  out_h.at[cid, pl.ds(sid * _ACC_SLICE, _ACC_SLICE)])


@functools.partial(
    pl.kernel,
    out_type=jax.ShapeDtypeStruct((_NC, _BATCH), jnp.float32),
    mesh=plsc.VectorSubcoreMesh(core_axis_name="c", subcore_axis_name="s",
                                num_cores=_NC, num_subcores=_NS),
    scratch_types=[
        pltpu.VMEM((_FEATURES,), jnp.float32),
        pltpu.VMEM((_CHUNK_R, _MINOR), jnp.int32),
        pltpu.VMEM((_CHUNK_R, _MINOR), jnp.float32),
        pltpu.VMEM((_CHUNK_R, _MINOR), jnp.int32),
        pltpu.VMEM((_CHUNK_R, _MINOR), jnp.int32),
        pltpu.VMEM((_CHUNK_R, _MINOR), jnp.float32),
        pltpu.VMEM((_ACC_SLICE,), jnp.float32),
        pltpu.VMEM((_NBANK, _ACC_SLICE), jnp.float32),
        pltpu.VMEM((_ACC_SLICE,), jnp.float32),
        pltpu.VMEM_SHARED((_NBANK * _BATCH,), jnp.float32),
        pltpu.SemaphoreType.DMA,
        pltpu.SemaphoreType.DMA,
        pltpu.SemaphoreType.DMA,
    ],
    compiler_params=pltpu.CompilerParams(needs_layout_passes=False),
)
def _sc_partials(row_h, col_h, val_h, w_h, out_h, *scratch):
  _sc_partial_kernel(row_h, col_h, val_h, w_h, out_h, *scratch)


def _combine_kernel(p_ref, o_ref):
  s = p_ref[0:1, :] + p_ref[1:2, :]
  o_ref[...] = jax.nn.sigmoid(s)


def kernel(row_idx, col_idx, values, W):
  row2d = row_idx.astype(jnp.int32).reshape(_ROWS_TOTAL, _MINOR)
  col2d = col_idx.astype(jnp.int32).reshape(_ROWS_TOTAL, _MINOR)
  val2d = values.reshape(_ROWS_TOTAL, _MINOR)

  partials = _sc_partials(row2d, col2d, val2d, W)

  logits = pl.pallas_call(
      _combine_kernel,
      out_shape=jax.ShapeDtypeStruct((1, _BATCH), jnp.float32),
  )(partials)
  return logits.reshape(_BATCH, 1)


# speedup vs baseline: 1.2617x; 1.0494x over previous
import functools

import jax
import jax.numpy as jnp
from jax import lax
from jax.experimental import pallas as pl
from jax.experimental.pallas import tpu as pltpu
from jax.experimental.pallas import tpu_sc as plsc

_BATCH = 16384
_FEATURES = 640
_NNZ = 655360

_NC = 2
_NS = 16
_LANES = 16
_NW = _NC * _NS

_MINOR = 128
_ROWS_TOTAL = _NNZ // _MINOR
_CHUNK_R = _ROWS_TOTAL // _NW
_ACC_SLICE = _BATCH // _NS
_FIRE_R = 8


def _sc_partial_kernel(row_h, col_h, val_h, w_h, out_h,
                       w_v, col_v, val_v, row_v, contrib, zero_v,
                       acc_sh, sem_in, sem_sc):
  cid = lax.axis_index("c")
  sid = lax.axis_index("s")
  base = (cid * _NS + sid) * _CHUNK_R

  cw = pltpu.async_copy(w_h.at[0], w_v, sem_in)
  cc = pltpu.async_copy(col_h.at[pl.ds(base, _CHUNK_R)], col_v, sem_in)
  cv = pltpu.async_copy(val_h.at[pl.ds(base, _CHUNK_R)], val_v, sem_in)
  cr = pltpu.async_copy(row_h.at[pl.ds(base, _CHUNK_R)], row_v, sem_in)

  @pl.loop(0, _ACC_SLICE // _LANES)
  def _zero(i):
    zero_v[pl.ds(i * _LANES, _LANES)] = jnp.zeros((_LANES,), jnp.float32)

  pltpu.sync_copy(zero_v, acc_sh.at[pl.ds(sid * _ACC_SLICE, _ACC_SLICE)])
  cw.wait()
  cc.wait()
  cv.wait()
  cr.wait()

  plsc.subcore_barrier()

  @pl.loop(0, _CHUNK_R // _FIRE_R)
  def _burst(b):
    r0 = b * _FIRE_R

    @plsc.parallel_loop(0, _FIRE_R * (_MINOR // _LANES))
    def _compute(g):
      r = r0 + g // (_MINOR // _LANES)
      sl = pl.ds((g % (_MINOR // _LANES)) * _LANES, _LANES)
      cols = col_v[r, sl]
      wv = plsc.load_gather(w_v, [cols])
      contrib[r, sl] = wv * val_v[r, sl]

    @pl.loop(r0, r0 + _FIRE_R)
    def _fire(r):
      pltpu.async_copy(contrib.at[r], acc_sh.at[row_v.at[r]], sem_sc,
                       add=True)

  @pl.loop(0, _CHUNK_R)
  def _drain(r):
    pltpu.make_async_copy(contrib.at[0], acc_sh.at[row_v.at[0]],
                          sem_sc).wait()

  plsc.subcore_barrier()

  pltpu.sync_copy(acc_sh.at[pl.ds(sid * _ACC_SLICE, _ACC_SLICE)],
                  out_h.at[cid, pl.ds(sid * _ACC_SLICE, _ACC_SLICE)])


@functools.partial(
    pl.kernel,
    out_type=jax.ShapeDtypeStruct((_NC, _BATCH), jnp.float32),
    mesh=plsc.VectorSubcoreMesh(core_axis_name="c", subcore_axis_name="s",
                                num_cores=_NC, num_subcores=_NS),
    scratch_types=[
        pltpu.VMEM((_FEATURES,), jnp.float32),
        pltpu.VMEM((_CHUNK_R, _MINOR), jnp.int32),
        pltpu.VMEM((_CHUNK_R, _MINOR), jnp.float32),
        pltpu.VMEM((_CHUNK_R, _MINOR), jnp.int32),
        pltpu.VMEM((_CHUNK_R, _MINOR), jnp.float32),
        pltpu.VMEM((_ACC_SLICE,), jnp.float32),
        pltpu.VMEM_SHARED((_BATCH,), jnp.float32),
        pltpu.SemaphoreType.DMA,
        pltpu.SemaphoreType.DMA,
    ],
    compiler_params=pltpu.CompilerParams(needs_layout_passes=False),
)
def _sc_partials(row_h, col_h, val_h, w_h, out_h, *scratch):
  _sc_partial_kernel(row_h, col_h, val_h, w_h, out_h, *scratch)


def _combine_kernel(p_ref, o_ref):
  s = p_ref[0:1, :] + p_ref[1:2, :]
  o_ref[...] = jax.nn.sigmoid(s)


def kernel(row_idx, col_idx, values, W):
  row2d = row_idx.astype(jnp.int32).reshape(_ROWS_TOTAL, _MINOR)
  col2d = col_idx.astype(jnp.int32).reshape(_ROWS_TOTAL, _MINOR)
  val2d = values.reshape(_ROWS_TOTAL, _MINOR)

  partials = _sc_partials(row2d, col2d, val2d, W)

  logits = pl.pallas_call(
      _combine_kernel,
      out_shape=jax.ShapeDtypeStruct((1, _BATCH), jnp.float32),
  )(partials)
  return logits.reshape(_BATCH, 1)
